# Initial kernel scaffold; baseline (speedup 1.0000x reference)
import jax
import jax.numpy as jnp
from jax.experimental import pallas as pl


def kernel(x, edge_index, edge_attr, W1, att_src1, att_dst1, b1, bn1_w, bn1_b, W2, att_src2, att_dst2, b2, bn2_w, bn2_b):
    raise NotImplementedError("write your pallas kernel here")


# speedup vs baseline: 22.4026x; 22.4026x over previous
import functools

import jax
import jax.numpy as jnp
from jax import lax
from jax.experimental import pallas as pl
from jax.experimental.pallas import tpu as pltpu
from jax.experimental.pallas import tpu_sc as plsc

N = 10000
E = 320000
ET = E + N
IN = 128
HID = 128
HEADS = 2
OUT = 128

NP_ = 10240
NSUB = 16
NCORE = 2
LANES = 16
C = 128
T = 20736
ETP = T * NSUB
NCH = T // C
RPT = NP_ // NSUB

RN = 256
RB = NP_ // RN


def _mm1_body(x_ref, w_ref, o_ref):
    o_ref[...] = jnp.dot(x_ref[...], w_ref[...],
                         preferred_element_type=jnp.float32)[None]


def _matmul1(x, W1):
    return pl.pallas_call(
        _mm1_body,
        grid=(HEADS, RB),
        in_specs=[
            pl.BlockSpec((RN, IN), lambda c, i: (i, 0)),
            pl.BlockSpec((IN, HID), lambda c, i: (0, c)),
        ],
        out_specs=pl.BlockSpec((1, RN, HID), lambda c, i: (c, i, 0)),
        out_shape=jax.ShapeDtypeStruct((HEADS, NP_, HID), jnp.float32),
    )(x, W1)


def _aux_body(x_ref, v_ref, o_ref):
    o_ref[...] = jax.lax.dot_general(
        v_ref[...], x_ref[...],
        dimension_numbers=(((1,), (1,)), ((), ())),
        preferred_element_type=jnp.float32)


def _aux1(x, V1):
    return pl.pallas_call(
        _aux_body,
        out_shape=jax.ShapeDtypeStruct((8, NP_), jnp.float32),
    )(x, V1)


def _mm2_body(h_ref, w_ref, b_ref, s_ref, t_ref, o_ref):
    h = h_ref[...]
    act0 = jnp.maximum(h[0] + b_ref[0], 0.0) * s_ref[0] + t_ref[0]
    act1 = jnp.maximum(h[1] + b_ref[1], 0.0) * s_ref[1] + t_ref[1]
    w = w_ref[...]
    acc = jnp.dot(act0, w[0:HID], preferred_element_type=jnp.float32)
    acc = acc + jnp.dot(act1, w[HID:2 * HID],
                        preferred_element_type=jnp.float32)
    o_ref[...] = acc


def _matmul2(h1, W2, b1r, s1r, t1r):
    return pl.pallas_call(
        _mm2_body,
        grid=(RB,),
        in_specs=[
            pl.BlockSpec((2, RN, HID), lambda i: (0, i, 0)),
            pl.BlockSpec((2 * HID, OUT), lambda i: (0, 0)),
            pl.BlockSpec((2, HID), lambda i: (0, 0)),
            pl.BlockSpec((2, HID), lambda i: (0, 0)),
            pl.BlockSpec((2, HID), lambda i: (0, 0)),
        ],
        out_specs=pl.BlockSpec((RN, OUT), lambda i: (i, 0)),
        out_shape=jax.ShapeDtypeStruct((NP_, OUT), jnp.float32),
    )(h1, W2, b1r, s1r, t1r)


def _aux2_body(h_ref, v_ref, b_ref, s_ref, t_ref, o_ref):
    h = h_ref[...]
    act0 = jnp.maximum(h[0] + b_ref[0], 0.0) * s_ref[0] + t_ref[0]
    act1 = jnp.maximum(h[1] + b_ref[1], 0.0) * s_ref[1] + t_ref[1]
    v = v_ref[...]
    a = jax.lax.dot_general(v[0], act0, (((1,), (1,)), ((), ())),
                            preferred_element_type=jnp.float32)
    a = a + jax.lax.dot_general(v[1], act1, (((1,), (1,)), ((), ())),
                                preferred_element_type=jnp.float32)
    o_ref[...] = a


def _aux2(h1, V2b, b1r, s1r, t1r):
    return pl.pallas_call(
        _aux2_body,
        out_shape=jax.ShapeDtypeStruct((8, NP_), jnp.float32),
    )(h1, V2b, b1r, s1r, t1r)


def _epi_body(h_ref, s_ref, t_ref, o_ref):
    h = h_ref[...]
    o_ref[...] = (h[0] + h[1]) * s_ref[...] + t_ref[...]


def _epilogue(h2, s2r, t2r):
    return pl.pallas_call(
        _epi_body,
        grid=(RB,),
        in_specs=[
            pl.BlockSpec((2, RN, OUT), lambda i: (0, i, 0)),
            pl.BlockSpec((1, OUT), lambda i: (0, 0)),
            pl.BlockSpec((1, OUT), lambda i: (0, 0)),
        ],
        out_specs=pl.BlockSpec((RN, OUT), lambda i: (i, 0)),
        out_shape=jax.ShapeDtypeStruct((NP_, OUT), jnp.float32),
    )(h2, s2r, t2r)


@functools.cache
def _make_sc_gat(layer1):
    F = 128
    FV = F // LANES
    mesh = plsc.VectorSubcoreMesh(core_axis_name="c", subcore_axis_name="s",
                                  num_cores=NCORE, num_subcores=NSUB)

    def body(h_ref, aux_ref, src_ref, dst_ref, o_ref,
             rows_v, asrc_v, adst_v, denom_v, srcc, dstc, coefc,
             gsrcc, acc_sp, den_sp, sem):
        cid = lax.axis_index("c")
        sid = lax.axis_index("s")
        z16 = jnp.zeros((LANES,), jnp.float32)

        def zrow(e, _):
            for j in range(FV):
                rows_v[e, pl.ds(j * LANES, LANES)] = z16
            return 0
        lax.fori_loop(0, C, zrow, 0)

        def zden(i, _):
            denom_v[pl.ds(i * LANES, LANES)] = z16
            return 0
        lax.fori_loop(0, NP_ // LANES, zden, 0)

        for k in range(RPT // C):
            pltpu.sync_copy(rows_v, acc_sp.at[pl.ds(sid * RPT + k * C, C)])
        pltpu.sync_copy(denom_v.at[pl.ds(0, RPT)],
                        den_sp.at[pl.ds(sid * RPT, RPT)])

        if layer1:
            pltpu.sync_copy(aux_ref.at[cid], asrc_v)
            pltpu.sync_copy(aux_ref.at[cid + 2], adst_v)
        else:
            pltpu.sync_copy(aux_ref.at[0], asrc_v)
            pltpu.sync_copy(aux_ref.at[1], adst_v)

        plsc.subcore_barrier()

        def p1(ch, _):
            base = sid * T + ch * C
            pltpu.sync_copy(src_ref.at[pl.ds(base, C)], srcc)
            pltpu.sync_copy(dst_ref.at[pl.ds(base, C)], dstc)
            for g in range(C // LANES):
                s16 = srcc[pl.ds(g * LANES, LANES)]
                d16 = dstc[pl.ds(g * LANES, LANES)]
                al = (plsc.load_gather(asrc_v, [s16])
                      + plsc.load_gather(adst_v, [d16]))
                al = jnp.maximum(al, 0.2 * al)
                ex = jnp.exp(al)
                glob = base + g * LANES + lax.iota(jnp.int32, LANES)
                ex = jnp.where(glob < ET, ex, 0.0)
                coefc[pl.ds(g * LANES, LANES)] = ex
            pltpu.sync_copy(coefc, den_sp.at[dstc], add=True)
            return 0
        lax.fori_loop(0, NCH, p1, 0)

        plsc.subcore_barrier()
        pltpu.sync_copy(den_sp, denom_v)

        if layer1:
            nch2 = NCH
        else:
            nch2 = NCH // 2

        def p2(ch, _):
            if layer1:
                base = sid * T + ch * C
            else:
                base = cid * (ETP // 2) + sid * (T // 2) + ch * C
            pltpu.sync_copy(src_ref.at[pl.ds(base, C)], srcc)
            pltpu.sync_copy(dst_ref.at[pl.ds(base, C)], dstc)
            if layer1:
                for g in range(C // LANES):
                    gsrcc[pl.ds(g * LANES, LANES)] = (
                        srcc[pl.ds(g * LANES, LANES)] + cid * NP_)
                pltpu.async_copy(h_ref.at[gsrcc], rows_v, sem).wait()
            else:
                pltpu.async_copy(h_ref.at[srcc], rows_v, sem).wait()
            for g in range(C // LANES):
                s16 = srcc[pl.ds(g * LANES, LANES)]
                d16 = dstc[pl.ds(g * LANES, LANES)]
                al = (plsc.load_gather(asrc_v, [s16])
                      + plsc.load_gather(adst_v, [d16]))
                al = jnp.maximum(al, 0.2 * al)
                ex = jnp.exp(al)
                glob = base + g * LANES + lax.iota(jnp.int32, LANES)
                ex = jnp.where(glob < ET, ex, 0.0)
                dn = plsc.load_gather(denom_v, [d16])
                coefc[pl.ds(g * LANES, LANES)] = ex / (dn + 1e-16)

            def scale(e, _):
                cv = plsc.load_gather(coefc, [jnp.full((LANES,), e,
                                                       jnp.int32)])
                for j in range(FV):
                    rv = rows_v[e, pl.ds(j * LANES, LANES)]
                    rows_v[e, pl.ds(j * LANES, LANES)] = rv * cv
                return 0
            lax.fori_loop(0, C, scale, 0)
            pltpu.sync_copy(rows_v, acc_sp.at[dstc], add=True)
            return 0
        lax.fori_loop(0, nch2, p2, 0)

        plsc.subcore_barrier()

        for k in range(RPT // C):
            r0 = sid * RPT + k * C
            pltpu.sync_copy(acc_sp.at[pl.ds(r0, C)], rows_v)
            pltpu.sync_copy(rows_v, o_ref.at[pl.ds(cid * NP_ + r0, C)])

    kern = pl.kernel(
        body,
        out_type=jax.ShapeDtypeStruct((NCORE * NP_, F), jnp.float32),
        mesh=mesh,
        compiler_params=pltpu.CompilerParams(needs_layout_passes=False),
        scratch_types=[
            pltpu.VMEM((C, F), jnp.float32),
            pltpu.VMEM((NP_,), jnp.float32),
            pltpu.VMEM((NP_,), jnp.float32),
            pltpu.VMEM((NP_,), jnp.float32),
            pltpu.VMEM((C,), jnp.int32),
            pltpu.VMEM((C,), jnp.int32),
            pltpu.VMEM((C,), jnp.float32),
            pltpu.VMEM((C,), jnp.int32),
            pltpu.VMEM_SHARED((NP_, F), jnp.float32),
            pltpu.VMEM_SHARED((NP_,), jnp.float32),
            pltpu.SemaphoreType.DMA,
        ],
    )
    return kern


def kernel(x, edge_index, edge_attr, W1, att_src1, att_dst1, b1, bn1_w,
           bn1_b, W2, att_src2, att_dst2, b2, bn2_w, bn2_b):
    f32 = jnp.float32
    xp = jnp.zeros((NP_, IN), f32).at[:N].set(x)
    loops = jnp.arange(N, dtype=edge_index.dtype)
    src = jnp.concatenate([edge_index[0], loops,
                           jnp.zeros((ETP - ET,), edge_index.dtype)])
    dst = jnp.concatenate([edge_index[1], loops,
                           jnp.zeros((ETP - ET,), edge_index.dtype)])

    W1h = W1.reshape(IN, HEADS, HID)
    V1 = jnp.zeros((8, IN), f32)
    for hh in range(HEADS):
        V1 = V1.at[hh].set(W1h[:, hh, :] @ att_src1[0, hh])
        V1 = V1.at[2 + hh].set(W1h[:, hh, :] @ att_dst1[0, hh])
    V2 = jnp.zeros((2, 8, HID), f32)
    vs2 = W2 @ att_src2[0, 0]
    vd2 = W2 @ att_dst2[0, 0]
    V2 = V2.at[0, 0].set(vs2[:HID]).at[1, 0].set(vs2[HID:])
    V2 = V2.at[0, 1].set(vd2[:HID]).at[1, 1].set(vd2[HID:])

    inv = 1.0 / jnp.sqrt(1.0 + 1e-5)
    b1r = b1.reshape(HEADS, HID)
    s1r = (bn1_w * inv).reshape(HEADS, HID)
    t1r = bn1_b.reshape(HEADS, HID)
    s2 = bn2_w * inv
    t2 = b2 * s2 + bn2_b
    s2r = s2.reshape(1, OUT)
    t2r = t2.reshape(1, OUT)

    h1 = _matmul1(xp, W1)
    aux1 = _aux1(xp, V1)
    o1 = _make_sc_gat(True)(h1.reshape(2 * NP_, HID), aux1, src, dst)
    o1 = o1.reshape(2, NP_, HID)

    h2 = _matmul2(o1, W2, b1r, s1r, t1r)
    aux2 = _aux2(o1, V2, b1r, s1r, t1r)
    o2 = _make_sc_gat(False)(h2, aux2, src, dst)
    o2 = o2.reshape(2, NP_, OUT)

    out = _epilogue(o2, s2r, t2r)
    return out[:N]
